# baseline (speedup 1.0000x reference)
import numpy as np
import jax
import jax.numpy as jnp
from jax import lax
from jax.experimental import pallas as pl
from jax.experimental.pallas import tpu as pltpu

NP = 128


def _make_body(T, C0, K1, K2, BB):
    S = C0 - K2 + 1

    def body(x_ref, wext_ref, g_ref, bf_ref, out_ref):
        sub = lax.broadcasted_iota(jnp.int32, (T, 128), 0)
        lane = lax.broadcasted_iota(jnp.int32, (T, 128), 1)
        valid = jnp.logical_or(sub < T - 1, lane < 128 - (K1 - 1))
        for bb in range(BB):
            sides = []
            for r in range(2):
                xb = x_ref[bb, r].astype(jnp.bfloat16)
                xup = jnp.concatenate(
                    [xb[1:], jnp.zeros((1, 128), jnp.bfloat16)], axis=0)
                xup = jnp.where(lane == K1 - 1, jnp.bfloat16(1), xup)
                xe = jnp.concatenate([xb, xup], axis=1)
                o = jnp.maximum(
                    jnp.dot(xe, wext_ref[r],
                            preferred_element_type=jnp.float32), 0.0)
                run = o[:, 0:128]
                cums = [run]
                for c in range(1, C0):
                    run = run + o[:, c * 128:(c + 1) * 128]
                    cums.append(run)
                sums = []
                for u in range(K2):
                    w = cums[u + S - 1]
                    if u > 0:
                        w = w - cums[u - 1]
                    sums.append(w)
                sides.append(sums)

            sw = [jnp.where(valid, s, 0.0).astype(jnp.bfloat16)
                  for s in sides[0]]
            sh = [s.astype(jnp.bfloat16) for s in sides[1]]

            acc = bf_ref[...]
            for u in range(K2):
                for v in range(K2):
                    p = jnp.sum(sw[u] * sh[v], axis=0, keepdims=True)
                    a_uv = jnp.sum(p, axis=1, keepdims=True
                                   ).astype(jnp.float32)
                    acc = acc + a_uv * g_ref[u * K2 + v:u * K2 + v + 1, :]
            out_ref[bb] = acc

    return body


def kernel(x, ww, bw, wh, bh, w2, b2, w1, b1, wlast, blast):
    B, _, _, L = x.shape
    C0, K1 = ww.shape
    K2 = w2.shape[1]
    S = C0 - K2 + 1
    NC = wlast.shape[0]
    LW = L - K1 + 1
    T = L // 128
    hp = lax.Precision.HIGHEST

    x2 = x[:, :, 0, :].reshape(B, 2, T, 128)

    e_np = np.zeros((K1, 256, 128), np.float32)
    for k in range(K1):
        e_np[k, np.arange(128) + k, np.arange(128)] = 1.0
    e = jnp.asarray(e_np)
    wstack = jnp.stack([ww, wh])
    wext = jnp.einsum('kJj,rck->rJcj', e, wstack, precision=hp)
    wext = wext.reshape(2, 256, C0 * 128)
    brow = jnp.repeat(jnp.stack([bw, bh]), 128, axis=-1)
    wext = wext.at[:, 128 + K1 - 1, :].set(brow).astype(jnp.bfloat16)

    g0 = jnp.einsum('cuv,mc,nm->uvn', w2, w1, wlast, precision=hp)
    g = jnp.zeros((K2 * K2, NP), jnp.float32)
    g = g.at[:, :NC].set(g0.reshape(K2 * K2, NC) / float(LW))
    bfold = (float(S * S) * jnp.einsum('c,mc,nm->n', b2, w1, wlast,
                                       precision=hp)
             + jnp.dot(wlast, b1, precision=hp) + blast)
    bfold = jnp.zeros((1, NP), jnp.float32).at[0, :NC].set(bfold)

    BB = 4
    out = pl.pallas_call(
        _make_body(T, C0, K1, K2, BB),
        out_shape=jax.ShapeDtypeStruct((B, 1, NP), jnp.float32),
        grid=(B // BB,),
        in_specs=[
            pl.BlockSpec((BB, 2, T, 128), lambda b: (b, 0, 0, 0)),
            pl.BlockSpec((2, 256, C0 * 128), lambda b: (0, 0, 0)),
            pl.BlockSpec((K2 * K2, NP), lambda b: (0, 0)),
            pl.BlockSpec((1, NP), lambda b: (0, 0)),
        ],
        out_specs=pl.BlockSpec((BB, 1, NP), lambda b: (b, 0, 0)),
        compiler_params=pltpu.CompilerParams(
            dimension_semantics=("parallel",)),
    )(x2, wext, g, bfold)
    return out[:, 0, :NC]


# speedup vs baseline: 1.0476x; 1.0476x over previous
import numpy as np
import jax
import jax.numpy as jnp
from jax import lax
from jax.experimental import pallas as pl
from jax.experimental.pallas import tpu as pltpu

NP = 128


def _make_body(T, C0, K1, K2, BB):
    S = C0 - K2 + 1

    def body(x_ref, wext_ref, g_ref, bf_ref, out_ref):
        sub = lax.broadcasted_iota(jnp.int32, (T, 128), 0)
        lane = lax.broadcasted_iota(jnp.int32, (T, 128), 1)
        valid = jnp.logical_or(sub < T - 1, lane < 128 - (K1 - 1))
        for bb in range(BB):
            sides = []
            for r in range(2):
                xb = x_ref[bb, r].astype(jnp.bfloat16)
                xup = jnp.concatenate(
                    [xb[1:], jnp.zeros((1, 128), jnp.bfloat16)], axis=0)
                xup = jnp.where(lane == K1 - 1, jnp.bfloat16(1), xup)
                xe = jnp.concatenate([xb, xup], axis=1)
                o = jnp.maximum(
                    jnp.dot(xe, wext_ref[r],
                            preferred_element_type=jnp.float32), 0.0)
                run = o[:, 0:128]
                cums = [run]
                for c in range(1, C0):
                    run = run + o[:, c * 128:(c + 1) * 128]
                    cums.append(run)
                sums = []
                for u in range(K2):
                    w = cums[u + S - 1]
                    if u > 0:
                        w = w - cums[u - 1]
                    sums.append(w)
                sides.append(sums)

            sw = [jnp.where(valid, s, 0.0) for s in sides[0]]
            sh = sides[1]

            acc = bf_ref[...]
            for u in range(K2):
                for v in range(K2):
                    p = jnp.sum(sw[u] * sh[v], axis=0, keepdims=True)
                    a_uv = jnp.sum(p, axis=1, keepdims=True)
                    acc = acc + a_uv * g_ref[u * K2 + v:u * K2 + v + 1, :]
            out_ref[bb] = acc

    return body


def kernel(x, ww, bw, wh, bh, w2, b2, w1, b1, wlast, blast):
    B, _, _, L = x.shape
    C0, K1 = ww.shape
    K2 = w2.shape[1]
    S = C0 - K2 + 1
    NC = wlast.shape[0]
    LW = L - K1 + 1
    T = L // 128
    hp = lax.Precision.HIGHEST

    x2 = x[:, :, 0, :].reshape(B, 2, T, 128)

    e_np = np.zeros((K1, 256, 128), np.float32)
    for k in range(K1):
        e_np[k, np.arange(128) + k, np.arange(128)] = 1.0
    e = jnp.asarray(e_np)
    wstack = jnp.stack([ww, wh])
    wext = jnp.einsum('kJj,rck->rJcj', e, wstack, precision=hp)
    wext = wext.reshape(2, 256, C0 * 128)
    brow = jnp.repeat(jnp.stack([bw, bh]), 128, axis=-1)
    wext = wext.at[:, 128 + K1 - 1, :].set(brow).astype(jnp.bfloat16)

    g0 = jnp.einsum('cuv,mc,nm->uvn', w2, w1, wlast, precision=hp)
    g = jnp.zeros((K2 * K2, NP), jnp.float32)
    g = g.at[:, :NC].set(g0.reshape(K2 * K2, NC) / float(LW))
    bfold = (float(S * S) * jnp.einsum('c,mc,nm->n', b2, w1, wlast,
                                       precision=hp)
             + jnp.dot(wlast, b1, precision=hp) + blast)
    bfold = jnp.zeros((1, NP), jnp.float32).at[0, :NC].set(bfold)

    BB = 8
    out = pl.pallas_call(
        _make_body(T, C0, K1, K2, BB),
        out_shape=jax.ShapeDtypeStruct((B, 1, NP), jnp.float32),
        grid=(B // BB,),
        in_specs=[
            pl.BlockSpec((BB, 2, T, 128), lambda b: (b, 0, 0, 0)),
            pl.BlockSpec((2, 256, C0 * 128), lambda b: (0, 0, 0)),
            pl.BlockSpec((K2 * K2, NP), lambda b: (0, 0)),
            pl.BlockSpec((1, NP), lambda b: (0, 0)),
        ],
        out_specs=pl.BlockSpec((BB, 1, NP), lambda b: (b, 0, 0)),
        compiler_params=pltpu.CompilerParams(
            dimension_semantics=("parallel",)),
    )(x2, wext, g, bfold)
    return out[:, 0, :NC]


# speedup vs baseline: 1.0525x; 1.0047x over previous
import numpy as np
import jax
import jax.numpy as jnp
from jax import lax
from jax.experimental import pallas as pl
from jax.experimental.pallas import tpu as pltpu

NP = 128


def _make_body(T, C0, K1, K2, BB):
    S = C0 - K2 + 1

    def body(x_ref, wext_ref, g_ref, bf_ref, out_ref):
        sub = lax.broadcasted_iota(jnp.int32, (T, 128), 0)
        lane = lax.broadcasted_iota(jnp.int32, (T, 128), 1)
        valid = jnp.logical_or(sub < T - 1, lane < 128 - (K1 - 1))
        for bb in range(BB):
            sides = []
            for r in range(2):
                xb = x_ref[bb, r].astype(jnp.bfloat16)
                xup = jnp.concatenate(
                    [xb[1:], jnp.zeros((1, 128), jnp.bfloat16)], axis=0)
                xup = jnp.where(lane == K1 - 1, jnp.bfloat16(1), xup)
                xe = jnp.concatenate([xb, xup], axis=1)
                o = jnp.maximum(
                    jnp.dot(xe, wext_ref[r],
                            preferred_element_type=jnp.float32), 0.0)
                run = o[:, 0:128]
                cums = [run]
                for c in range(1, C0):
                    run = run + o[:, c * 128:(c + 1) * 128]
                    cums.append(run)
                sums = []
                for u in range(K2):
                    w = cums[u + S - 1]
                    if u > 0:
                        w = w - cums[u - 1]
                    sums.append(w)
                sides.append(sums)

            sw = [jnp.where(valid, s, 0.0) for s in sides[0]]
            sh = sides[1]

            acc = bf_ref[...]
            for u in range(K2):
                for v in range(K2):
                    p = jnp.sum(sw[u] * sh[v], axis=0, keepdims=True)
                    a_uv = jnp.sum(p, axis=1, keepdims=True)
                    acc = acc + a_uv * g_ref[u * K2 + v:u * K2 + v + 1, :]
            out_ref[bb] = acc

    return body


def kernel(x, ww, bw, wh, bh, w2, b2, w1, b1, wlast, blast):
    B, _, _, L = x.shape
    C0, K1 = ww.shape
    K2 = w2.shape[1]
    S = C0 - K2 + 1
    NC = wlast.shape[0]
    LW = L - K1 + 1
    T = L // 128
    hp = lax.Precision.HIGHEST

    x2 = x[:, :, 0, :].reshape(B, 2, T, 128)

    e_np = np.zeros((K1, 256, 128), np.float32)
    for k in range(K1):
        e_np[k, np.arange(128) + k, np.arange(128)] = 1.0
    e = jnp.asarray(e_np)
    wstack = jnp.stack([ww, wh])
    wext = jnp.einsum('kJj,rck->rJcj', e, wstack, precision=hp)
    wext = wext.reshape(2, 256, C0 * 128)
    brow = jnp.repeat(jnp.stack([bw, bh]), 128, axis=-1)
    wext = wext.at[:, 128 + K1 - 1, :].set(brow).astype(jnp.bfloat16)

    g0 = jnp.einsum('cuv,mc,nm->uvn', w2, w1, wlast, precision=hp)
    g = jnp.zeros((K2 * K2, NP), jnp.float32)
    g = g.at[:, :NC].set(g0.reshape(K2 * K2, NC) / float(LW))
    bfold = (float(S * S) * jnp.einsum('c,mc,nm->n', b2, w1, wlast,
                                       precision=hp)
             + jnp.dot(wlast, b1, precision=hp) + blast)
    bfold = jnp.zeros((1, NP), jnp.float32).at[0, :NC].set(bfold)

    BB = 8
    out = pl.pallas_call(
        _make_body(T, C0, K1, K2, BB),
        out_shape=jax.ShapeDtypeStruct((B, 1, NP), jnp.float32),
        grid=(B // BB,),
        in_specs=[
            pl.BlockSpec((BB, 2, T, 128), lambda b: (b, 0, 0, 0)),
            pl.BlockSpec((2, 256, C0 * 128), lambda b: (0, 0, 0)),
            pl.BlockSpec((K2 * K2, NP), lambda b: (0, 0)),
            pl.BlockSpec((1, NP), lambda b: (0, 0)),
        ],
        out_specs=pl.BlockSpec((BB, 1, NP), lambda b: (b, 0, 0)),
        compiler_params=pltpu.CompilerParams(
            dimension_semantics=("parallel",),
            flags={"XLA_TPU_STORE_TO_LOAD_FORWARDING_WINDOW": 12288}),
    )(x2, wext, g, bfold)
    return out[:, 0, :NC]
